# Initial kernel scaffold; baseline (speedup 1.0000x reference)
import jax
import jax.numpy as jnp
from jax.experimental import pallas as pl


def kernel(cat_a, cat_b, cat_c, W_cat_a, W_cat_b, W_cat_c):
    raise NotImplementedError("write your pallas kernel here")


# speedup vs baseline: 1.9892x; 1.9892x over previous
import functools

import jax
import jax.numpy as jnp
from jax import lax
from jax.experimental import pallas as pl
from jax.experimental.pallas import tpu as pltpu
from jax.experimental.pallas import tpu_sc as plsc

_D = 64
_N = 16384
_C = 20
_TOT = _N * _C
_NW = 32
_PER_W = _TOT // _NW
_CH = 128
_NCHUNK = _PER_W // _CH


def _make_kernel():
    mesh = plsc.VectorSubcoreMesh(core_axis_name="c", subcore_axis_name="s")
    out_t = [jax.ShapeDtypeStruct((_TOT, _D), jnp.float32) for _ in range(3)]
    scratch = [
        pltpu.VMEM((_CH,), jnp.int32),
        pltpu.VMEM((_CH, _D), jnp.float32),
        pltpu.SemaphoreType.DMA,
    ]

    @functools.partial(
        pl.kernel, out_type=out_t, mesh=mesh, scratch_types=scratch,
        compiler_params=pltpu.CompilerParams(use_tc_tiling_on_sc=False))
    def k(ia, ib, ic, wa, wb, wc, oa, ob, oc, idx_v, rows_v, sem):
        wid = lax.axis_index("s") * 2 + lax.axis_index("c")
        base = wid * _PER_W
        for idx_hbm, tab_hbm, out_hbm in ((ia, wa, oa), (ib, wb, ob),
                                          (ic, wc, oc)):
            def body(j, _, idx_hbm=idx_hbm, tab_hbm=tab_hbm,
                     out_hbm=out_hbm):
                off = base + j * _CH
                pltpu.sync_copy(idx_hbm.at[pl.ds(off, _CH)], idx_v)
                pltpu.async_copy(tab_hbm.at[idx_v], rows_v, sem).wait()
                pltpu.sync_copy(rows_v, out_hbm.at[pl.ds(off, _CH)])
                return 0
            lax.fori_loop(0, _NCHUNK, body, 0)

    return k


_gather3 = jax.jit(_make_kernel())


def kernel(cat_a, cat_b, cat_c, W_cat_a, W_cat_b, W_cat_c):
    ia = cat_a.reshape(-1)
    ib = cat_b.reshape(-1)
    ic = cat_c.reshape(-1)
    oa, ob, oc = _gather3(ia, ib, ic, W_cat_a, W_cat_b, W_cat_c)
    shape = (_N, _C, _D)
    return (oa.reshape(shape), ob.reshape(shape), oc.reshape(shape))


# speedup vs baseline: 2.3054x; 1.1590x over previous
import functools

import jax
import jax.numpy as jnp
from jax import lax
from jax.experimental import pallas as pl
from jax.experimental.pallas import tpu as pltpu
from jax.experimental.pallas import tpu_sc as plsc

_D = 64
_N = 16384
_C = 20
_TOT = _N * _C
_NW = 32
_PER_W = _TOT // _NW
_CH = 1024
_NCHUNK = _PER_W // _CH


def _make_kernel():
    mesh = plsc.VectorSubcoreMesh(core_axis_name="c", subcore_axis_name="s")
    out_t = [jax.ShapeDtypeStruct((_TOT, _D), jnp.float32) for _ in range(3)]
    scratch = [
        pltpu.VMEM((_CH,), jnp.int32),
        pltpu.VMEM((_CH, _D), jnp.float32),
        pltpu.SemaphoreType.DMA,
    ]

    @functools.partial(
        pl.kernel, out_type=out_t, mesh=mesh, scratch_types=scratch,
        compiler_params=pltpu.CompilerParams(use_tc_tiling_on_sc=False))
    def k(ia, ib, ic, wa, wb, wc, oa, ob, oc, idx_v, rows_v, sem):
        wid = lax.axis_index("s") * 2 + lax.axis_index("c")
        base = wid * _PER_W
        for idx_hbm, tab_hbm, out_hbm in ((ia, wa, oa), (ib, wb, ob),
                                          (ic, wc, oc)):
            def body(j, _, idx_hbm=idx_hbm, tab_hbm=tab_hbm,
                     out_hbm=out_hbm):
                off = base + j * _CH
                pltpu.sync_copy(idx_hbm.at[pl.ds(off, _CH)], idx_v)
                pltpu.async_copy(tab_hbm.at[idx_v], rows_v, sem).wait()
                pltpu.sync_copy(rows_v, out_hbm.at[pl.ds(off, _CH)])
                return 0
            lax.fori_loop(0, _NCHUNK, body, 0)

    return k


_gather3 = jax.jit(_make_kernel())


def kernel(cat_a, cat_b, cat_c, W_cat_a, W_cat_b, W_cat_c):
    ia = cat_a.reshape(-1)
    ib = cat_b.reshape(-1)
    ic = cat_c.reshape(-1)
    oa, ob, oc = _gather3(ia, ib, ic, W_cat_a, W_cat_b, W_cat_c)
    shape = (_N, _C, _D)
    return (oa.reshape(shape), ob.reshape(shape), oc.reshape(shape))


# speedup vs baseline: 2.3338x; 1.0123x over previous
import functools

import jax
import jax.numpy as jnp
from jax import lax
from jax.experimental import pallas as pl
from jax.experimental.pallas import tpu as pltpu
from jax.experimental.pallas import tpu_sc as plsc

_D = 64
_N = 16384
_C = 20
_TOT = _N * _C
_NW = 32
_PER_W = _TOT // _NW
_CH = 512
_NCH = _PER_W // _CH
_NB = 2
_NGRP = _NCH // _NB


def _make_kernel():
    mesh = plsc.VectorSubcoreMesh(core_axis_name="c", subcore_axis_name="s")
    out_t = [jax.ShapeDtypeStruct((_TOT, _D), jnp.float32) for _ in range(3)]
    scratch = [
        pltpu.VMEM((_PER_W,), jnp.int32),
        pltpu.VMEM((_CH, _D), jnp.float32),
        pltpu.VMEM((_CH, _D), jnp.float32),
        pltpu.SemaphoreType.DMA,
        pltpu.SemaphoreType.DMA,
    ]

    @functools.partial(
        pl.kernel, out_type=out_t, mesh=mesh, scratch_types=scratch,
        compiler_params=pltpu.CompilerParams(use_tc_tiling_on_sc=False))
    def k(ia, ib, ic, wa, wb, wc, oa, ob, oc, idxall, rows0, rows1,
          gsem, wsem):
        wid = lax.axis_index("s") * 2 + lax.axis_index("c")
        base = wid * _PER_W
        rows = (rows0, rows1)

        for idx_hbm, tab_hbm, out_hbm in ((ia, wa, oa), (ib, wb, ob),
                                          (ic, wc, oc)):
            def g_fire(j, b, tab=tab_hbm):
                pltpu.async_copy(
                    tab.at[idxall.at[pl.ds(j * _CH, _CH)]], rows[b], gsem)

            def g_wait(b, tab=tab_hbm):
                pltpu.make_async_copy(
                    tab.at[idxall.at[pl.ds(0, _CH)]], rows[b], gsem).wait()

            def w_fire(j, b, out=out_hbm):
                pltpu.async_copy(
                    rows[b], out.at[pl.ds(base + j * _CH, _CH)], wsem)

            def w_wait(b, out=out_hbm):
                pltpu.make_async_copy(
                    rows[b], out.at[pl.ds(base, _CH)], wsem).wait()

            pltpu.sync_copy(idx_hbm.at[pl.ds(base, _PER_W)], idxall)
            g_fire(0, 0)
            g_fire(1, 1)
            g_wait(0)
            w_fire(0, 0)
            w_wait(0)
            g_fire(2, 0)
            g_wait(1)
            w_fire(1, 1)

            def grp(g, _):
                for b in range(_NB):
                    j = g * _NB + b
                    w_wait((b + 1) % _NB)
                    g_fire(j + 1, (b + 1) % _NB)
                    g_wait(b)
                    w_fire(j, b)
                return 0

            lax.fori_loop(1, _NGRP - 1, grp, 0)

            w_wait(1)
            g_fire(_NCH - 1, 1)
            g_wait(0)
            w_fire(_NCH - 2, 0)
            w_wait(0)
            g_wait(1)
            w_fire(_NCH - 1, 1)
            w_wait(1)

    return k


_gather3 = jax.jit(_make_kernel())


def kernel(cat_a, cat_b, cat_c, W_cat_a, W_cat_b, W_cat_c):
    ia = cat_a.reshape(-1)
    ib = cat_b.reshape(-1)
    ic = cat_c.reshape(-1)
    oa, ob, oc = _gather3(ia, ib, ic, W_cat_a, W_cat_b, W_cat_c)
    shape = (_N, _C, _D)
    return (oa.reshape(shape), ob.reshape(shape), oc.reshape(shape))
